# Initial kernel scaffold; baseline (speedup 1.0000x reference)
import jax
import jax.numpy as jnp
from jax.experimental import pallas as pl


def kernel(x, adj, W1, W2):
    raise NotImplementedError("write your pallas kernel here")


# speedup vs baseline: 1.0236x; 1.0236x over previous
import functools

import jax
import jax.numpy as jnp
from jax.experimental import pallas as pl
from jax.experimental.pallas import tpu as pltpu

_BM = 400


def _gcn_kernel(x_ref, w1_ref, w2_ref, adj_ref, out_ref, s1_ref, s2_ref, *, nb):
    i = pl.program_id(0)

    @pl.when(i == 0)
    def _prologue():
        xb = x_ref[...].astype(jnp.bfloat16)
        w1b = w1_ref[...].astype(jnp.bfloat16)
        s1 = jnp.dot(xb, w1b, preferred_element_type=jnp.float32)
        s1_ref[...] = s1.astype(jnp.bfloat16)

    adj_b = adj_ref[...].astype(jnp.bfloat16)

    @pl.when(i < nb)
    def _layer1():
        h = jnp.dot(adj_b, s1_ref[...], preferred_element_type=jnp.float32)
        h = jnp.maximum(h, 0.0).astype(jnp.bfloat16)
        w2b = w2_ref[...].astype(jnp.bfloat16)
        s2 = jnp.dot(h, w2b, preferred_element_type=jnp.float32)
        s2_ref[pl.ds((i % nb) * _BM, _BM), :] = s2.astype(jnp.bfloat16)

    @pl.when(i >= nb)
    def _layer2():
        out_ref[...] = jnp.dot(adj_b, s2_ref[...],
                               preferred_element_type=jnp.float32)


@jax.jit
def kernel(x, adj, W1, W2):
    n, nfeat = x.shape
    nhid = W1.shape[1]
    nout = W2.shape[1]
    nb = n // _BM

    grid = (2 * nb,)
    return pl.pallas_call(
        functools.partial(_gcn_kernel, nb=nb),
        grid=grid,
        in_specs=[
            pl.BlockSpec((n, nfeat), lambda i: (0, 0)),
            pl.BlockSpec((nfeat, nhid), lambda i: (0, 0)),
            pl.BlockSpec((nhid, nout), lambda i: (0, 0)),
            pl.BlockSpec((_BM, n), lambda i, nb=nb: (i % nb, 0)),
        ],
        out_specs=pl.BlockSpec((_BM, nout), lambda i, nb=nb: (i % nb, 0)),
        out_shape=jax.ShapeDtypeStruct((n, nout), jnp.float32),
        scratch_shapes=[
            pltpu.VMEM((n, nhid), jnp.bfloat16),
            pltpu.VMEM((n, nout), jnp.bfloat16),
        ],
        compiler_params=pltpu.CompilerParams(
            vmem_limit_bytes=100 * 1024 * 1024,
        ),
    )(x, W1, W2, adj)


# speedup vs baseline: 1.0237x; 1.0001x over previous
import functools

import jax
import jax.numpy as jnp
from jax.experimental import pallas as pl
from jax.experimental.pallas import tpu as pltpu

_BM = 400


def _gcn_kernel(x_ref, w1_ref, w2_ref, adj_ref, out_ref, s1_ref, s2_ref, *, nb):
    i = pl.program_id(0)

    @pl.when(i == 0)
    def _prologue():
        xb = x_ref[...].astype(jnp.bfloat16)
        w1b = w1_ref[...].astype(jnp.bfloat16)
        s1 = jnp.dot(xb, w1b, preferred_element_type=jnp.float32)
        s1_ref[...] = s1.astype(jnp.bfloat16)

    adj_b = adj_ref[...].astype(jnp.bfloat16)

    @pl.when(i < nb)
    def _layer1():
        h = jnp.dot(adj_b, s1_ref[...], preferred_element_type=jnp.float32)
        h = jnp.maximum(h, 0.0).astype(jnp.bfloat16)
        w2b = w2_ref[...].astype(jnp.bfloat16)
        s2 = jnp.dot(h, w2b, preferred_element_type=jnp.float32)
        s2_ref[pl.ds((i % nb) * _BM, _BM), :] = s2.astype(jnp.bfloat16)

    @pl.when(i >= nb)
    def _layer2():
        out_ref[...] = jnp.dot(adj_b, s2_ref[...],
                               preferred_element_type=jnp.float32)


@jax.jit
def kernel(x, adj, W1, W2):
    n, nfeat = x.shape
    nhid = W1.shape[1]
    nout = W2.shape[1]
    nb = n // _BM

    grid = (2 * nb,)
    return pl.pallas_call(
        functools.partial(_gcn_kernel, nb=nb),
        grid=grid,
        in_specs=[
            pl.BlockSpec((n, nfeat), lambda i: (0, 0)),
            pl.BlockSpec((nfeat, nhid), lambda i: (0, 0)),
            pl.BlockSpec((nhid, nout), lambda i: (0, 0)),
            pl.BlockSpec((_BM, n), lambda i, nb=nb: (i % nb, 0)),
        ],
        out_specs=pl.BlockSpec(
            (_BM, nout),
            lambda i, nb=nb: (jnp.where(i >= nb, i - nb, 0), 0)),
        out_shape=jax.ShapeDtypeStruct((n, nout), jnp.float32),
        scratch_shapes=[
            pltpu.VMEM((n, nhid), jnp.bfloat16),
            pltpu.VMEM((n, nout), jnp.bfloat16),
        ],
        compiler_params=pltpu.CompilerParams(
            vmem_limit_bytes=100 * 1024 * 1024,
        ),
    )(x, W1, W2, adj)
